# baseline (speedup 1.0000x reference)
import functools

import jax
import jax.numpy as jnp
from jax import lax
from jax.experimental import pallas as pl
from jax.experimental.pallas import tpu as pltpu
from jax.experimental.pallas import tpu_sc as plsc

_NC = 2
_NS = 16
_B = 128
_G = 64


@functools.cache
def _make_agg(n_pad: int, w: int, cpt: int, split: bool):
  nw = _NC * _NS
  rpt = n_pad // _NS
  my_cpt = cpt * _NC if split else cpt
  nb = 4
  assert my_cpt % nb == 0 and my_cpt >= 2 * nb
  mesh = plsc.VectorSubcoreMesh(core_axis_name="c", subcore_axis_name="s")

  @functools.partial(
      pl.kernel,
      mesh=mesh,
      out_type=jax.ShapeDtypeStruct((_NC, n_pad, w), jnp.float32),
      scratch_types=[
          pltpu.VMEM((my_cpt, _B), jnp.int32),
          pltpu.VMEM((my_cpt, _B), jnp.int32),
          [pltpu.VMEM((_B, w), jnp.float32) for _ in range(nb)],
          [pltpu.SemaphoreType.DMA for _ in range(nb)],
          [pltpu.SemaphoreType.DMA for _ in range(nb)],
          pltpu.VMEM_SHARED((n_pad, w), jnp.float32),
          pltpu.SemaphoreType.DMA,
          pltpu.SemaphoreType.DMA,
          pltpu.SemaphoreType.DMA,
      ],
      compiler_params=pltpu.CompilerParams(use_tc_tiling_on_sc=False),
  )
  def agg(u_hbm, src_hbm, dst_hbm, zeros_hbm, out_hbm,
          sidx, didx, rows, gsem, ssem, acc, isem0, isem1, zsem):
    c = lax.axis_index("c")
    s = lax.axis_index("s")
    u_view = u_hbm.at[c]
    if split:
      base_chunk = s * my_cpt
    else:
      base_chunk = (c * _NS + s) * my_cpt
    icp0 = pltpu.async_copy(src_hbm.at[pl.ds(base_chunk, my_cpt)], sidx, isem0)
    icp1 = pltpu.async_copy(dst_hbm.at[pl.ds(base_chunk, my_cpt)], didx, isem1)
    zrows = zeros_hbm.at[pl.ds(s * rpt, rpt)]
    zcp = pltpu.async_copy(zrows, acc.at[pl.ds(s * rpt, rpt)], zsem)
    icp0.wait()
    icp1.wait()
    for b in range(nb):
      pltpu.async_copy(u_view.at[sidx.at[b]], rows[b], gsem[b])
    zcp.wait()
    plsc.subcore_barrier()

    def body(j, carry):
      for b in range(nb):
        i = nb * j + b
        bp = (b - 1) % nb
        ip = i - 1
        pltpu.make_async_copy(u_view.at[sidx.at[i]], rows[b], gsem[b]).wait()
        pltpu.async_copy(rows[b], acc.at[didx.at[i]], ssem[b], add=True)

        @pl.when(ip >= 0)
        def _():
          pltpu.make_async_copy(
              rows[bp], acc.at[didx.at[ip]], ssem[bp]).wait()

          @pl.when(ip + nb < my_cpt)
          def _():
            pltpu.async_copy(u_view.at[sidx.at[ip + nb]], rows[bp], gsem[bp])

      return carry

    lax.fori_loop(0, my_cpt // nb, body, 0)
    pltpu.make_async_copy(
        rows[nb - 1], acc.at[didx.at[my_cpt - 1]], ssem[nb - 1]).wait()
    plsc.subcore_barrier()
    pltpu.sync_copy(acc.at[pl.ds(s * rpt, rpt)],
                    out_hbm.at[c, pl.ds(s * rpt, rpt)])

  return agg


def _tc_xw1(x_ref, w1_ref, xw_ref):
  xw_ref[...] = jnp.dot(x_ref[...], w1_ref[...],
                        preferred_element_type=jnp.float32)


def _tc_scale1(xw_ref, degp_ref, u1_ref, dis_ref, *, n):
  d = degp_ref[...]
  n_pad = d.shape[1]
  deg = d[0, :, 0:1] + d[1, :, 0:1] + 1.0
  mask = (lax.broadcasted_iota(jnp.int32, (n_pad, 1), 0) < n).astype(
      jnp.float32)
  dis = lax.rsqrt(deg) * mask
  dis_ref[...] = dis
  u1 = dis * xw_ref[...]
  u1_ref[0] = u1
  u1_ref[1] = u1


def _tc_epilogue1(acc_ref, u_ref, dis_ref, b_ref, out_ref):
  a = acc_ref[...]
  dis = dis_ref[...]
  h = jnp.maximum(dis * (a[0] + a[1] + u_ref[0]) + b_ref[...], 0.0)
  u2 = dis * h
  out_ref[0] = u2
  out_ref[1] = u2


def _tc_layer2(acc_ref, u_ref, dis_ref, w_ref, b_ref, out_ref):
  a = acc_ref[...]
  dis = dis_ref[...]
  z = dis * (a[0] + a[1] + u_ref[0])
  zw = jnp.dot(z, w_ref[...], preferred_element_type=jnp.float32)
  v = dis * jnp.maximum(zw + b_ref[...], 0.0)
  hw = v.shape[1] // 2
  out_ref[0] = v[:, :hw]
  out_ref[1] = v[:, hw:]


def _tc_layer3(acc_ref, u_ref, dis_ref, w_ref, b_ref, out_ref):
  a = acc_ref[...]
  uu = u_ref[...]
  dis = dis_ref[...]
  z = dis * jnp.concatenate([a[0] + uu[0], a[1] + uu[1]], axis=1)
  zw = jnp.dot(z, w_ref[...], preferred_element_type=jnp.float32)
  v = dis * jnp.maximum(zw + b_ref[...], 0.0)
  hw = v.shape[1] // 2
  out_ref[0] = v[:, :hw]
  out_ref[1] = v[:, hw:]


def _tc_head(acc_ref, u_ref, dis_ref, w4_ref, b4_ref, batch_ref,
             l1_ref, lb1_ref, l2_ref, lb2_ref, y_ref):
  a = acc_ref[...]
  uu = u_ref[...]
  dis = dis_ref[...]
  z = dis * jnp.concatenate([a[0] + uu[0], a[1] + uu[1]], axis=1)
  zw = jnp.dot(z, w4_ref[...], preferred_element_type=jnp.float32)
  h4 = jnp.maximum(zw + b4_ref[...], 0.0)
  n_pad = h4.shape[0]
  gids = lax.broadcasted_iota(jnp.int32, (_G, n_pad), 0)
  onehot = (batch_ref[...] == gids).astype(jnp.float32)
  sums = jnp.dot(onehot, h4, preferred_element_type=jnp.float32)
  cnt = jnp.sum(onehot, axis=1, keepdims=True)
  pooled = sums / jnp.maximum(cnt, 1.0)
  t = jnp.maximum(
      jnp.dot(pooled, l1_ref[...], preferred_element_type=jnp.float32)
      + lb1_ref[...], 0.0)
  y_ref[...] = (jnp.dot(t, l2_ref[...], preferred_element_type=jnp.float32)
                + lb2_ref[...])


def _call(body, out_shapes, *args):
  return pl.pallas_call(body, out_shape=out_shapes)(*args)


@jax.jit
def kernel(x, edge_index, batch, W1, b1, W2, b2, W3, b3, W4, b4,
           L1, lb1, L2, lb2):
  n, d = x.shape
  e = edge_index.shape[1]
  n_pad = ((n + 1 + 127) // 128) * 128
  nw = _NC * _NS
  cpt = (e + nw * _B - 1) // (nw * _B)
  cpt = ((cpt + 3) // 4) * 4
  e_pad = nw * _B * cpt

  f32 = jnp.float32
  x_pad = jnp.zeros((n_pad, d), f32).at[:n].set(x)
  src2d = jnp.full((e_pad,), n, jnp.int32).at[:e].set(edge_index[0])
  src2d = src2d.reshape(e_pad // _B, _B)
  spread = (jnp.arange(e_pad - e, dtype=jnp.int32)) % n
  dst2d = jnp.concatenate([edge_index[1], spread])
  dst2d = dst2d.reshape(e_pad // _B, _B)
  batch2d = jnp.full((1, n_pad), _G, jnp.int32).at[0, :n].set(batch)

  ones1 = jnp.zeros((n_pad, 16), f32).at[:n].set(1.0)
  ones16 = jnp.stack([ones1, ones1])
  zeros_of = {w: jnp.zeros((n_pad, w), f32) for w in (16, 32, 64)}

  def agg(u, w, split):
    return _make_agg(n_pad, w, cpt, split)(u, src2d, dst2d, zeros_of[w])

  sds = jax.ShapeDtypeStruct
  degp = agg(ones16, 16, False)

  w1o = W1.shape[1]
  xw1 = _call(_tc_xw1, sds((n_pad, w1o), f32), x_pad, W1)
  u1, dis = _call(
      functools.partial(_tc_scale1, n=n),
      [sds((2, n_pad, w1o), f32), sds((n_pad, 1), f32)],
      xw1, degp)

  acc1 = agg(u1, w1o, False)
  u2 = _call(_tc_epilogue1, sds((2, n_pad, w1o), f32),
             acc1, u1, dis, b1.reshape(1, -1))

  acc2 = agg(u2, w1o, False)
  w2o = W2.shape[1]
  u3 = _call(_tc_layer2, sds((2, n_pad, w2o // 2), f32),
             acc2, u2, dis, W2, b2.reshape(1, -1))

  acc3 = agg(u3, w2o // 2, True)
  w3o = W3.shape[1]
  u4 = _call(_tc_layer3, sds((2, n_pad, w3o // 2), f32),
             acc3, u3, dis, W3, b3.reshape(1, -1))

  acc4 = agg(u4, w3o // 2, True)
  y = _call(_tc_head, sds((_G, L2.shape[1]), f32),
            acc4, u4, dis, W4, b4.reshape(1, -1), batch2d,
            L1, lb1.reshape(1, -1), L2, lb2.reshape(1, -1))
  return y


# speedup vs baseline: 1.1624x; 1.1624x over previous
import functools

import jax
import jax.numpy as jnp
from jax import lax
from jax.experimental import pallas as pl
from jax.experimental.pallas import tpu as pltpu
from jax.experimental.pallas import tpu_sc as plsc

_NC = 2
_NS = 16
_B = 128
_G = 64


@functools.cache
def _make_agg(n_pad: int, w: int, cpt: int, split: bool):
  nw = _NC * _NS
  rpt = n_pad // _NS
  my_cpt = cpt * _NC if split else cpt
  nb = 4
  assert my_cpt % nb == 0 and my_cpt >= 2 * nb
  mesh = plsc.VectorSubcoreMesh(core_axis_name="c", subcore_axis_name="s")

  @functools.partial(
      pl.kernel,
      mesh=mesh,
      out_type=jax.ShapeDtypeStruct((_NC, n_pad, w), jnp.float32),
      scratch_types=[
          pltpu.VMEM((my_cpt, _B), jnp.int32),
          pltpu.VMEM((my_cpt, _B), jnp.int32),
          [pltpu.VMEM((_B, w), jnp.float32) for _ in range(nb)],
          [pltpu.SemaphoreType.DMA for _ in range(nb)],
          [pltpu.SemaphoreType.DMA for _ in range(nb)],
          pltpu.VMEM_SHARED((n_pad, w), jnp.float32),
          pltpu.SemaphoreType.DMA,
          pltpu.SemaphoreType.DMA,
          pltpu.SemaphoreType.DMA,
      ],
      compiler_params=pltpu.CompilerParams(use_tc_tiling_on_sc=False),
  )
  def agg(u_hbm, src_hbm, dst_hbm, zeros_hbm, out_hbm,
          sidx, didx, rows, gsem, ssem, acc, isem0, isem1, zsem):
    c = lax.axis_index("c")
    s = lax.axis_index("s")
    if split:
      base_chunk = s * my_cpt
      u_view = u_hbm.at[c]
    else:
      base_chunk = (c * _NS + s) * my_cpt
      u_view = u_hbm
    icp0 = pltpu.async_copy(src_hbm.at[pl.ds(base_chunk, my_cpt)], sidx, isem0)
    icp1 = pltpu.async_copy(dst_hbm.at[pl.ds(base_chunk, my_cpt)], didx, isem1)
    zrows = zeros_hbm.at[pl.ds(s * rpt, rpt)]
    zcp = pltpu.async_copy(zrows, acc.at[pl.ds(s * rpt, rpt)], zsem)
    icp0.wait()
    icp1.wait()
    for b in range(nb):
      pltpu.async_copy(u_view.at[sidx.at[b]], rows[b], gsem[b])
    zcp.wait()
    plsc.subcore_barrier()

    def body(j, carry):
      for b in range(nb):
        i = nb * j + b
        bp = (b - 1) % nb
        ip = i - 1
        pltpu.make_async_copy(u_view.at[sidx.at[i]], rows[b], gsem[b]).wait()
        pltpu.async_copy(rows[b], acc.at[didx.at[i]], ssem[b], add=True)

        @pl.when(ip >= 0)
        def _():
          pltpu.make_async_copy(
              rows[bp], acc.at[didx.at[ip]], ssem[bp]).wait()

          @pl.when(ip + nb < my_cpt)
          def _():
            pltpu.async_copy(u_view.at[sidx.at[ip + nb]], rows[bp], gsem[bp])

      return carry

    lax.fori_loop(0, my_cpt // nb, body, 0)
    pltpu.make_async_copy(
        rows[nb - 1], acc.at[didx.at[my_cpt - 1]], ssem[nb - 1]).wait()
    plsc.subcore_barrier()
    pltpu.sync_copy(acc.at[pl.ds(s * rpt, rpt)],
                    out_hbm.at[c, pl.ds(s * rpt, rpt)])

  return agg


def _tc_xw1(x_ref, w1_ref, xw_ref):
  xw_ref[...] = jnp.dot(x_ref[...], w1_ref[...],
                        preferred_element_type=jnp.float32)


def _tc_scale1(xw_ref, degp_ref, u1_ref, dis_ref, *, n):
  d = degp_ref[...]
  n_pad = d.shape[1]
  deg = d[0, :, 0:1] + d[1, :, 0:1] + 1.0
  mask = (lax.broadcasted_iota(jnp.int32, (n_pad, 1), 0) < n).astype(
      jnp.float32)
  dis = lax.rsqrt(deg) * mask
  dis_ref[...] = dis
  u1_ref[...] = dis * xw_ref[...]


def _tc_epilogue1(acc_ref, u_ref, dis_ref, b_ref, out_ref):
  a = acc_ref[...]
  dis = dis_ref[...]
  h = jnp.maximum(dis * (a[0] + a[1] + u_ref[...]) + b_ref[...], 0.0)
  out_ref[...] = dis * h


def _tc_layer2(acc_ref, u_ref, dis_ref, w_ref, b_ref, out_ref):
  a = acc_ref[...]
  dis = dis_ref[...]
  z = dis * (a[0] + a[1] + u_ref[...])
  zw = jnp.dot(z, w_ref[...], preferred_element_type=jnp.float32)
  v = dis * jnp.maximum(zw + b_ref[...], 0.0)
  hw = v.shape[1] // 2
  out_ref[0] = v[:, :hw]
  out_ref[1] = v[:, hw:]


def _tc_layer3(acc_ref, u_ref, dis_ref, w_ref, b_ref, out_ref):
  a = acc_ref[...]
  uu = u_ref[...]
  dis = dis_ref[...]
  z = dis * jnp.concatenate([a[0] + uu[0], a[1] + uu[1]], axis=1)
  zw = jnp.dot(z, w_ref[...], preferred_element_type=jnp.float32)
  v = dis * jnp.maximum(zw + b_ref[...], 0.0)
  hw = v.shape[1] // 2
  out_ref[0] = v[:, :hw]
  out_ref[1] = v[:, hw:]


def _tc_head(acc_ref, u_ref, dis_ref, w4_ref, b4_ref, batch_ref,
             l1_ref, lb1_ref, l2_ref, lb2_ref, y_ref):
  a = acc_ref[...]
  uu = u_ref[...]
  dis = dis_ref[...]
  z = dis * jnp.concatenate([a[0] + uu[0], a[1] + uu[1]], axis=1)
  zw = jnp.dot(z, w4_ref[...], preferred_element_type=jnp.float32)
  h4 = jnp.maximum(zw + b4_ref[...], 0.0)
  n_pad = h4.shape[0]
  gids = lax.broadcasted_iota(jnp.int32, (_G, n_pad), 0)
  onehot = (batch_ref[...] == gids).astype(jnp.float32)
  sums = jnp.dot(onehot, h4, preferred_element_type=jnp.float32)
  cnt = jnp.sum(onehot, axis=1, keepdims=True)
  pooled = sums / jnp.maximum(cnt, 1.0)
  t = jnp.maximum(
      jnp.dot(pooled, l1_ref[...], preferred_element_type=jnp.float32)
      + lb1_ref[...], 0.0)
  y_ref[...] = (jnp.dot(t, l2_ref[...], preferred_element_type=jnp.float32)
                + lb2_ref[...])


def _call(body, out_shapes, *args):
  return pl.pallas_call(body, out_shape=out_shapes)(*args)


@jax.jit
def kernel(x, edge_index, batch, W1, b1, W2, b2, W3, b3, W4, b4,
           L1, lb1, L2, lb2):
  n, d = x.shape
  e = edge_index.shape[1]
  n_pad = ((n + 1 + 127) // 128) * 128
  nw = _NC * _NS
  cpt = (e + nw * _B - 1) // (nw * _B)
  cpt = ((cpt + 3) // 4) * 4
  e_pad = nw * _B * cpt

  f32 = jnp.float32
  x_pad = jnp.zeros((n_pad, d), f32).at[:n].set(x)
  nchunks = e_pad // _B
  rpc = -(-e // nchunks)
  e_r = nchunks * rpc
  i32 = jnp.int32
  src_r = jnp.full((e_r,), n, i32).at[:e].set(edge_index[0])
  src2d = jnp.concatenate(
      [src_r.reshape(nchunks, rpc),
       jnp.full((nchunks, _B - rpc), n, i32)], axis=1)
  spread_r = jnp.arange(e_r, dtype=i32) % n
  dst_r = spread_r.at[:e].set(edge_index[1])
  spread_p = (jnp.arange(nchunks * (_B - rpc), dtype=i32) % n)
  dst2d = jnp.concatenate(
      [dst_r.reshape(nchunks, rpc),
       spread_p.reshape(nchunks, _B - rpc)], axis=1)
  batch2d = jnp.full((1, n_pad), _G, jnp.int32).at[0, :n].set(batch)

  ones16 = jnp.zeros((n_pad, 16), f32).at[:n].set(1.0)
  zeros_of = {w: jnp.zeros((n_pad, w), f32) for w in (16, 32, 64)}

  def agg(u, w, split):
    return _make_agg(n_pad, w, cpt, split)(u, src2d, dst2d, zeros_of[w])

  sds = jax.ShapeDtypeStruct
  degp = agg(ones16, 16, False)

  w1o = W1.shape[1]
  xw1 = _call(_tc_xw1, sds((n_pad, w1o), f32), x_pad, W1)
  u1, dis = _call(
      functools.partial(_tc_scale1, n=n),
      [sds((n_pad, w1o), f32), sds((n_pad, 1), f32)],
      xw1, degp)

  acc1 = agg(u1, w1o, False)
  u2 = _call(_tc_epilogue1, sds((n_pad, w1o), f32),
             acc1, u1, dis, b1.reshape(1, -1))

  acc2 = agg(u2, w1o, False)
  w2o = W2.shape[1]
  u3 = _call(_tc_layer2, sds((2, n_pad, w2o // 2), f32),
             acc2, u2, dis, W2, b2.reshape(1, -1))

  acc3 = agg(u3, w2o // 2, True)
  w3o = W3.shape[1]
  u4 = _call(_tc_layer3, sds((2, n_pad, w3o // 2), f32),
             acc3, u3, dis, W3, b3.reshape(1, -1))

  acc4 = agg(u4, w3o // 2, True)
  y = _call(_tc_head, sds((_G, L2.shape[1]), f32),
            acc4, u4, dis, W4, b4.reshape(1, -1), batch2d,
            L1, lb1.reshape(1, -1), L2, lb2.reshape(1, -1))
  return y


# speedup vs baseline: 1.4508x; 1.2481x over previous
import functools

import jax
import jax.numpy as jnp
from jax import lax
from jax.experimental import pallas as pl
from jax.experimental.pallas import tpu as pltpu
from jax.experimental.pallas import tpu_sc as plsc

_NC = 2
_NS = 16
_B = 128
_G = 64


@functools.cache
def _make_agg(n_pad: int, w: int, cpt: int, split: bool, reps: int):
  nw = _NC * _NS
  rpt = n_pad // _NS
  my_cpt = cpt * _NC if split else cpt
  nb = 4
  assert my_cpt % nb == 0 and my_cpt >= 2 * nb
  mesh = plsc.VectorSubcoreMesh(core_axis_name="c", subcore_axis_name="s")

  @functools.partial(
      pl.kernel,
      mesh=mesh,
      out_type=jax.ShapeDtypeStruct((_NC, n_pad, w), jnp.float32),
      scratch_types=[
          pltpu.VMEM((my_cpt, _B), jnp.int32),
          pltpu.VMEM((my_cpt, _B), jnp.int32),
          [pltpu.VMEM((_B, w), jnp.float32) for _ in range(nb)],
          [pltpu.SemaphoreType.DMA for _ in range(nb)],
          [pltpu.SemaphoreType.DMA for _ in range(nb)],
          pltpu.VMEM_SHARED((n_pad, w), jnp.float32),
          pltpu.SemaphoreType.DMA,
          pltpu.SemaphoreType.DMA,
          pltpu.SemaphoreType.DMA,
      ],
      compiler_params=pltpu.CompilerParams(use_tc_tiling_on_sc=False),
  )
  def agg(u_hbm, src_hbm, dst_hbm, zeros_hbm, out_hbm,
          sidx, didx, rows, gsem, ssem, acc, isem0, isem1, zsem):
    c = lax.axis_index("c")
    s = lax.axis_index("s")
    if split:
      base_chunk = s * my_cpt
      u_view = u_hbm.at[c].at[s % reps]
    else:
      base_chunk = (c * _NS + s) * my_cpt
      u_view = u_hbm.at[(c * _NS + s) % reps]
    icp0 = pltpu.async_copy(src_hbm.at[pl.ds(base_chunk, my_cpt)], sidx, isem0)
    icp1 = pltpu.async_copy(dst_hbm.at[pl.ds(base_chunk, my_cpt)], didx, isem1)
    zrows = zeros_hbm.at[pl.ds(s * rpt, rpt)]
    zcp = pltpu.async_copy(zrows, acc.at[pl.ds(s * rpt, rpt)], zsem)
    icp0.wait()
    icp1.wait()
    for b in range(nb):
      pltpu.async_copy(u_view.at[sidx.at[b]], rows[b], gsem[b])
    zcp.wait()
    plsc.subcore_barrier()

    def body(j, carry):
      for b in range(nb):
        i = nb * j + b
        bp = (b - 1) % nb
        ip = i - 1
        pltpu.make_async_copy(u_view.at[sidx.at[i]], rows[b], gsem[b]).wait()
        pltpu.async_copy(rows[b], acc.at[didx.at[i]], ssem[b], add=True)

        @pl.when(ip >= 0)
        def _():
          pltpu.make_async_copy(
              rows[bp], acc.at[didx.at[ip]], ssem[bp]).wait()

          @pl.when(ip + nb < my_cpt)
          def _():
            pltpu.async_copy(u_view.at[sidx.at[ip + nb]], rows[bp], gsem[bp])

      return carry

    lax.fori_loop(0, my_cpt // nb, body, 0)
    pltpu.make_async_copy(
        rows[nb - 1], acc.at[didx.at[my_cpt - 1]], ssem[nb - 1]).wait()
    plsc.subcore_barrier()
    pltpu.sync_copy(acc.at[pl.ds(s * rpt, rpt)],
                    out_hbm.at[c, pl.ds(s * rpt, rpt)])

  return agg


def _tc_xw1(x_ref, w1_ref, xw_ref):
  xw_ref[...] = jnp.dot(x_ref[...], w1_ref[...],
                        preferred_element_type=jnp.float32)


def _tc_scale1(xw_ref, degp_ref, u1_ref, dis_ref, *, n):
  d = degp_ref[...]
  n_pad = d.shape[1]
  deg = d[0, :, 0:1] + d[1, :, 0:1] + 1.0
  mask = (lax.broadcasted_iota(jnp.int32, (n_pad, 1), 0) < n).astype(
      jnp.float32)
  dis = lax.rsqrt(deg) * mask
  dis_ref[...] = dis
  u1 = dis * xw_ref[...]
  for r in range(u1_ref.shape[0]):
    u1_ref[r] = u1


def _tc_epilogue1(acc_ref, u_ref, dis_ref, b_ref, out_ref):
  a = acc_ref[...]
  dis = dis_ref[...]
  h = jnp.maximum(dis * (a[0] + a[1] + u_ref[0]) + b_ref[...], 0.0)
  u2 = dis * h
  for r in range(out_ref.shape[0]):
    out_ref[r] = u2


def _tc_layer2(acc_ref, u_ref, dis_ref, w_ref, b_ref, out_ref):
  a = acc_ref[...]
  dis = dis_ref[...]
  z = dis * (a[0] + a[1] + u_ref[0])
  zw = jnp.dot(z, w_ref[...], preferred_element_type=jnp.float32)
  v = dis * jnp.maximum(zw + b_ref[...], 0.0)
  hw = v.shape[1] // 2
  for r in range(out_ref.shape[1]):
    out_ref[0, r] = v[:, :hw]
    out_ref[1, r] = v[:, hw:]


def _tc_layer3(acc_ref, u_ref, dis_ref, w_ref, b_ref, out_ref):
  a = acc_ref[...]
  dis = dis_ref[...]
  z = dis * jnp.concatenate([a[0] + u_ref[0, 0], a[1] + u_ref[1, 0]], axis=1)
  zw = jnp.dot(z, w_ref[...], preferred_element_type=jnp.float32)
  v = dis * jnp.maximum(zw + b_ref[...], 0.0)
  hw = v.shape[1] // 2
  for r in range(out_ref.shape[1]):
    out_ref[0, r] = v[:, :hw]
    out_ref[1, r] = v[:, hw:]


def _tc_head(acc_ref, u_ref, dis_ref, w4_ref, b4_ref, batch_ref,
             l1_ref, lb1_ref, l2_ref, lb2_ref, y_ref):
  a = acc_ref[...]
  dis = dis_ref[...]
  z = dis * jnp.concatenate([a[0] + u_ref[0, 0], a[1] + u_ref[1, 0]], axis=1)
  zw = jnp.dot(z, w4_ref[...], preferred_element_type=jnp.float32)
  h4 = jnp.maximum(zw + b4_ref[...], 0.0)
  n_pad = h4.shape[0]
  gids = lax.broadcasted_iota(jnp.int32, (_G, n_pad), 0)
  onehot = (batch_ref[...] == gids).astype(jnp.float32)
  sums = jnp.dot(onehot, h4, preferred_element_type=jnp.float32)
  cnt = jnp.sum(onehot, axis=1, keepdims=True)
  pooled = sums / jnp.maximum(cnt, 1.0)
  t = jnp.maximum(
      jnp.dot(pooled, l1_ref[...], preferred_element_type=jnp.float32)
      + lb1_ref[...], 0.0)
  y_ref[...] = (jnp.dot(t, l2_ref[...], preferred_element_type=jnp.float32)
                + lb2_ref[...])


def _call(body, out_shapes, *args):
  return pl.pallas_call(
      body, out_shape=out_shapes,
      compiler_params=pltpu.CompilerParams(
          vmem_limit_bytes=100 * 1024 * 1024))(*args)


@jax.jit
def kernel(x, edge_index, batch, W1, b1, W2, b2, W3, b3, W4, b4,
           L1, lb1, L2, lb2):
  n, d = x.shape
  e = edge_index.shape[1]
  n_pad = ((n + 1 + 127) // 128) * 128
  nw = _NC * _NS
  cpt = (e + nw * _B - 1) // (nw * _B)
  cpt = ((cpt + 3) // 4) * 4
  e_pad = nw * _B * cpt

  f32 = jnp.float32
  x_pad = jnp.zeros((n_pad, d), f32).at[:n].set(x)
  nchunks = e_pad // _B
  rpc = -(-e // nchunks)
  e_r = nchunks * rpc
  i32 = jnp.int32
  src_r = jnp.full((e_r,), n, i32).at[:e].set(edge_index[0])
  src2d = jnp.concatenate(
      [src_r.reshape(nchunks, rpc),
       jnp.full((nchunks, _B - rpc), n, i32)], axis=1)
  spread_r = jnp.arange(e_r, dtype=i32) % n
  dst_r = spread_r.at[:e].set(edge_index[1])
  spread_p = (jnp.arange(nchunks * (_B - rpc), dtype=i32) % n)
  dst2d = jnp.concatenate(
      [dst_r.reshape(nchunks, rpc),
       spread_p.reshape(nchunks, _B - rpc)], axis=1)
  batch2d = jnp.full((1, n_pad), _G, jnp.int32).at[0, :n].set(batch)

  _REPS = 2
  _REPSD = 4
  _REPS2 = 2
  ones1 = jnp.zeros((n_pad, 16), f32).at[:n].set(1.0)
  ones16 = jnp.broadcast_to(ones1, (_REPSD, n_pad, 16))
  zeros_of = {w: jnp.zeros((n_pad, w), f32) for w in (16, 32, 64)}

  def agg(u, w, split, reps=None):
    if reps is None:
      reps = _REPS2 if split else _REPS
    return _make_agg(n_pad, w, cpt, split, reps)(u, src2d, dst2d, zeros_of[w])

  sds = jax.ShapeDtypeStruct
  degp = agg(ones16, 16, False, reps=_REPSD)

  w1o = W1.shape[1]
  xw1 = _call(_tc_xw1, sds((n_pad, w1o), f32), x_pad, W1)
  u1, dis = _call(
      functools.partial(_tc_scale1, n=n),
      [sds((2, n_pad, w1o), f32), sds((n_pad, 1), f32)],
      xw1, degp)

  acc1 = agg(u1, w1o, False)
  u2 = _call(_tc_epilogue1, sds((2, n_pad, w1o), f32),
             acc1, u1, dis, b1.reshape(1, -1))

  acc2 = agg(u2, w1o, False)
  w2o = W2.shape[1]
  u3 = _call(_tc_layer2, sds((2, 2, n_pad, w2o // 2), f32),
             acc2, u2, dis, W2, b2.reshape(1, -1))

  acc3 = agg(u3, w2o // 2, True)
  w3o = W3.shape[1]
  u4 = _call(_tc_layer3, sds((2, 2, n_pad, w3o // 2), f32),
             acc3, u3, dis, W3, b3.reshape(1, -1))

  acc4 = agg(u4, w3o // 2, True)
  y = _call(_tc_head, sds((_G, L2.shape[1]), f32),
            acc4, u4, dis, W4, b4.reshape(1, -1), batch2d,
            L1, lb1.reshape(1, -1), L2, lb2.reshape(1, -1))
  return y


# speedup vs baseline: 1.4643x; 1.0093x over previous
import functools

import jax
import jax.numpy as jnp
from jax import lax
from jax.experimental import pallas as pl
from jax.experimental.pallas import tpu as pltpu
from jax.experimental.pallas import tpu_sc as plsc

_NC = 2
_NS = 16
_B = 128
_G = 64


@functools.cache
def _make_agg(n_pad: int, w: int, cpt: int, split: bool, reps: int):
  nw = _NC * _NS
  rpt = n_pad // _NS
  my_cpt = cpt * _NC if split else cpt
  nb = 4
  assert my_cpt % nb == 0 and my_cpt >= 2 * nb
  mesh = plsc.VectorSubcoreMesh(core_axis_name="c", subcore_axis_name="s")

  @functools.partial(
      pl.kernel,
      mesh=mesh,
      out_type=jax.ShapeDtypeStruct((_NC, n_pad, w), jnp.float32),
      scratch_types=[
          pltpu.VMEM((my_cpt, _B), jnp.int32),
          pltpu.VMEM((my_cpt, _B), jnp.int32),
          [pltpu.VMEM((_B, w), jnp.float32) for _ in range(nb)],
          [pltpu.SemaphoreType.DMA for _ in range(nb)],
          [pltpu.SemaphoreType.DMA for _ in range(nb)],
          pltpu.VMEM_SHARED((n_pad, w), jnp.float32),
          pltpu.SemaphoreType.DMA,
          pltpu.SemaphoreType.DMA,
          pltpu.SemaphoreType.DMA,
      ],
      compiler_params=pltpu.CompilerParams(use_tc_tiling_on_sc=False),
  )
  def agg(u_hbm, src_hbm, dst_hbm, zeros_hbm, out_hbm,
          sidx, didx, rows, gsem, ssem, acc, isem0, isem1, zsem):
    c = lax.axis_index("c")
    s = lax.axis_index("s")
    if split:
      base_chunk = s * my_cpt
      u_view = u_hbm.at[c].at[s % reps]
    else:
      base_chunk = (c * _NS + s) * my_cpt
      u_view = u_hbm.at[(c * _NS + s) % reps]
    icp0 = pltpu.async_copy(src_hbm.at[pl.ds(base_chunk, my_cpt)], sidx, isem0)
    icp1 = pltpu.async_copy(dst_hbm.at[pl.ds(base_chunk, my_cpt)], didx, isem1)
    zrows = zeros_hbm.at[pl.ds(s * rpt, rpt)]
    zcp = pltpu.async_copy(zrows, acc.at[pl.ds(s * rpt, rpt)], zsem)
    icp0.wait()
    icp1.wait()
    for b in range(nb):
      pltpu.async_copy(u_view.at[sidx.at[b]], rows[b], gsem[b])
    zcp.wait()
    plsc.subcore_barrier()

    def body(j, carry):
      for b in range(nb):
        i = nb * j + b
        bp = (b - 1) % nb
        ip = i - 1
        pltpu.make_async_copy(u_view.at[sidx.at[i]], rows[b], gsem[b]).wait()
        pltpu.async_copy(rows[b], acc.at[didx.at[i]], ssem[b], add=True)

        @pl.when(ip >= 0)
        def _():
          pltpu.make_async_copy(
              rows[bp], acc.at[didx.at[ip]], ssem[bp]).wait()

          @pl.when(ip + nb < my_cpt)
          def _():
            pltpu.async_copy(u_view.at[sidx.at[ip + nb]], rows[bp], gsem[bp])

      return carry

    lax.fori_loop(0, my_cpt // nb, body, 0)
    pltpu.make_async_copy(
        rows[nb - 1], acc.at[didx.at[my_cpt - 1]], ssem[nb - 1]).wait()
    plsc.subcore_barrier()
    pltpu.sync_copy(acc.at[pl.ds(s * rpt, rpt)],
                    out_hbm.at[c, pl.ds(s * rpt, rpt)])

  return agg


def _tc_xw1(x_ref, w1_ref, xw_ref):
  xw_ref[...] = jnp.dot(x_ref[...], w1_ref[...],
                        preferred_element_type=jnp.float32)


def _tc_scale1(xw_ref, degp_ref, u1_ref, dis_ref, *, n):
  d = degp_ref[...]
  n_pad = d.shape[1]
  deg = d[0, :, 0:1] + d[1, :, 0:1] + 1.0
  mask = (lax.broadcasted_iota(jnp.int32, (n_pad, 1), 0) < n).astype(
      jnp.float32)
  dis = lax.rsqrt(deg) * mask
  dis_ref[...] = dis
  u1 = dis * xw_ref[...]
  for r in range(u1_ref.shape[0]):
    u1_ref[r] = u1


def _tc_epilogue1(acc_ref, u_ref, dis_ref, b_ref, out_ref):
  a = acc_ref[...]
  dis = dis_ref[...]
  h = jnp.maximum(dis * (a[0] + a[1] + u_ref[0]) + b_ref[...], 0.0)
  u2 = dis * h
  for r in range(out_ref.shape[0]):
    out_ref[r] = u2


def _tc_layer2(acc_ref, u_ref, dis_ref, w_ref, b_ref, out_ref):
  a = acc_ref[...]
  dis = dis_ref[...]
  z = dis * (a[0] + a[1] + u_ref[0])
  zw = jnp.dot(z, w_ref[...], preferred_element_type=jnp.float32)
  v = dis * jnp.maximum(zw + b_ref[...], 0.0)
  hw = v.shape[1] // 2
  for r in range(out_ref.shape[1]):
    out_ref[0, r] = v[:, :hw]
    out_ref[1, r] = v[:, hw:]


def _tc_layer3(acc_ref, u_ref, dis_ref, w_ref, b_ref, out_ref):
  a = acc_ref[...]
  dis = dis_ref[...]
  z = dis * jnp.concatenate([a[0] + u_ref[0, 0], a[1] + u_ref[1, 0]], axis=1)
  zw = jnp.dot(z, w_ref[...], preferred_element_type=jnp.float32)
  v = dis * jnp.maximum(zw + b_ref[...], 0.0)
  hw = v.shape[1] // 2
  for r in range(out_ref.shape[1]):
    out_ref[0, r] = v[:, :hw]
    out_ref[1, r] = v[:, hw:]


def _tc_head(acc_ref, u_ref, dis_ref, w4_ref, b4_ref, batch_ref,
             l1_ref, lb1_ref, l2_ref, lb2_ref, y_ref):
  a = acc_ref[...]
  dis = dis_ref[...]
  z = dis * jnp.concatenate([a[0] + u_ref[0, 0], a[1] + u_ref[1, 0]], axis=1)
  zw = jnp.dot(z, w4_ref[...], preferred_element_type=jnp.float32)
  h4 = jnp.maximum(zw + b4_ref[...], 0.0)
  n_pad = h4.shape[0]
  gids = lax.broadcasted_iota(jnp.int32, (_G, n_pad), 0)
  onehot = (batch_ref[...] == gids).astype(jnp.float32)
  sums = jnp.dot(onehot, h4, preferred_element_type=jnp.float32)
  cnt = jnp.sum(onehot, axis=1, keepdims=True)
  pooled = sums / jnp.maximum(cnt, 1.0)
  t = jnp.maximum(
      jnp.dot(pooled, l1_ref[...], preferred_element_type=jnp.float32)
      + lb1_ref[...], 0.0)
  y_ref[...] = (jnp.dot(t, l2_ref[...], preferred_element_type=jnp.float32)
                + lb2_ref[...])


def _call(body, out_shapes, *args):
  return pl.pallas_call(
      body, out_shape=out_shapes,
      compiler_params=pltpu.CompilerParams(
          vmem_limit_bytes=100 * 1024 * 1024))(*args)


@jax.jit
def kernel(x, edge_index, batch, W1, b1, W2, b2, W3, b3, W4, b4,
           L1, lb1, L2, lb2):
  n, d = x.shape
  e = edge_index.shape[1]
  n_pad = ((n + 1 + 127) // 128) * 128
  nw = _NC * _NS
  cpt = (e + nw * _B - 1) // (nw * _B)
  cpt = ((cpt + 3) // 4) * 4
  e_pad = nw * _B * cpt

  f32 = jnp.float32
  x_pad = jnp.zeros((n_pad, d), f32).at[:n].set(x)
  nchunks = e_pad // _B
  rpc = -(-e // nchunks)
  e_r = nchunks * rpc
  i32 = jnp.int32
  src_r = jnp.full((e_r,), n, i32).at[:e].set(edge_index[0])
  src2d = jnp.concatenate(
      [src_r.reshape(nchunks, rpc),
       jnp.full((nchunks, _B - rpc), n, i32)], axis=1)
  spread_r = jnp.arange(e_r, dtype=i32) % n
  dst_r = spread_r.at[:e].set(edge_index[1])
  spread_p = (jnp.arange(nchunks * (_B - rpc), dtype=i32) % n)
  dst2d = jnp.concatenate(
      [dst_r.reshape(nchunks, rpc),
       spread_p.reshape(nchunks, _B - rpc)], axis=1)
  batch2d = jnp.full((1, n_pad), _G, jnp.int32).at[0, :n].set(batch)

  _REPS = 4
  _REPSD = 4
  ones1 = jnp.zeros((n_pad, 16), f32).at[:n].set(1.0)
  ones16 = jnp.broadcast_to(ones1, (_REPSD, n_pad, 16))
  zeros_of = {w: jnp.zeros((n_pad, w), f32) for w in (16, 32, 64)}

  def agg(u, w, split, reps=_REPS):
    return _make_agg(n_pad, w, cpt, split, reps)(u, src2d, dst2d, zeros_of[w])

  sds = jax.ShapeDtypeStruct
  degp = agg(ones16, 16, False, reps=_REPSD)

  w1o = W1.shape[1]
  xw1 = _call(_tc_xw1, sds((n_pad, w1o), f32), x_pad, W1)
  u1, dis = _call(
      functools.partial(_tc_scale1, n=n),
      [sds((4, n_pad, w1o), f32), sds((n_pad, 1), f32)],
      xw1, degp)

  acc1 = agg(u1, w1o, False)
  u2 = _call(_tc_epilogue1, sds((4, n_pad, w1o), f32),
             acc1, u1, dis, b1.reshape(1, -1))

  acc2 = agg(u2, w1o, False)
  w2o = W2.shape[1]
  u3 = _call(_tc_layer2, sds((2, 2, n_pad, w2o // 2), f32),
             acc2, u2, dis, W2, b2.reshape(1, -1))

  acc3 = agg(u3, w2o // 2, True, reps=2)
  w3o = W3.shape[1]
  u4 = _call(_tc_layer3, sds((2, 2, n_pad, w3o // 2), f32),
             acc3, u3, dis, W3, b3.reshape(1, -1))

  acc4 = agg(u4, w3o // 2, True, reps=2)
  y = _call(_tc_head, sds((_G, L2.shape[1]), f32),
            acc4, u4, dis, W4, b4.reshape(1, -1), batch2d,
            L1, lb1.reshape(1, -1), L2, lb2.reshape(1, -1))
  return y
